# Initial kernel scaffold; baseline (speedup 1.0000x reference)
import jax
import jax.numpy as jnp
from jax.experimental import pallas as pl


def kernel(x, edge_index, edge_weight, init_weight, weight, root_weight, bias):
    raise NotImplementedError("write your pallas kernel here")


# speedup vs baseline: 17.2388x; 17.2388x over previous
import functools
import jax
import jax.numpy as jnp
from jax import lax
from jax.experimental import pallas as pl
from jax.experimental.pallas import tpu as pltpu
from jax.experimental.pallas import tpu_sc as plsc

N = 10000
E = 160000
F = 256
HALF = 128
K = 2
NC = 2
NS = 16
L = 16

G = 80
SB = 8
NSB = 16
SBT = NSB + 1
NG = NSB * SB
EP = NG * G
E_PAD = NS * EP
EW_DEG = E_PAD // (NC * NS)
NH = 10240
NP = 10112
NPT = NP // NS

BN = 1000
NB = N // BN

_MESH = plsc.VectorSubcoreMesh(
    core_axis_name="c", subcore_axis_name="s", num_cores=NC, num_subcores=NS)
_SC_PARAMS = pltpu.CompilerParams(needs_layout_passes=False)


@functools.partial(
    pl.kernel,
    out_type=jax.ShapeDtypeStruct((NC * NS * NH,), jnp.float32),
    mesh=_MESH,
    scratch_types=[
        pltpu.VMEM((EW_DEG,), jnp.int32),
        pltpu.VMEM((EW_DEG,), jnp.float32),
        pltpu.VMEM((NH,), jnp.float32),
    ],
    compiler_params=_SC_PARAMS,
)
def _deg_kernel(col_hbm, ew_hbm, out_hbm, col_v, ew_v, acc_v):
    c = lax.axis_index("c")
    s = lax.axis_index("s")
    wid = c * NS + s
    base = wid * EW_DEG
    pltpu.sync_copy(col_hbm.at[pl.ds(base, EW_DEG)], col_v)
    pltpu.sync_copy(ew_hbm.at[pl.ds(base, EW_DEG)], ew_v)

    zeros = jnp.zeros((L,), jnp.float32)

    @pl.loop(0, NH // L)
    def _zero(i):
        acc_v[pl.ds(i * L, L)] = zeros

    @pl.loop(0, EW_DEG // L)
    def _accum(g):
        cv = col_v[pl.ds(g * L, L)]
        wv = ew_v[pl.ds(g * L, L)]
        plsc.addupdate_scatter(acc_v, [cv], wv)

    pltpu.sync_copy(acc_v, out_hbm.at[pl.ds(wid * NH, NH)])


@functools.partial(
    pl.kernel,
    out_type=jax.ShapeDtypeStruct((K * NC * NP, HALF), jnp.float32),
    mesh=_MESH,
    scratch_types=[
        pltpu.VMEM((2 * SB, G), jnp.int32),
        pltpu.VMEM((2 * SB, G), jnp.int32),
        pltpu.VMEM((2 * SB * G,), jnp.float32),
        pltpu.VMEM((G, HALF), jnp.float32),
        pltpu.VMEM((G, HALF), jnp.float32),
        pltpu.VMEM((G, HALF), jnp.float32),
        pltpu.VMEM((G, HALF), jnp.float32),
        pltpu.VMEM_SHARED((NP, HALF), jnp.float32),
        pltpu.SemaphoreType.DMA,
        pltpu.SemaphoreType.DMA,
        pltpu.SemaphoreType.DMA,
        pltpu.SemaphoreType.DMA,
        pltpu.SemaphoreType.DMA,
    ],
    compiler_params=_SC_PARAMS,
)
def _prop_kernel(hs_hbm, rows_hbm, cols_hbm, ew_hbm, zeros_hbm, out_hbm,
                 rows_v, cols_v, ew_v, stage0, stage1, scaled0, scaled1,
                 acc_sh, gsem0, gsem1, ssem0, ssem1, lsem):
    c = lax.axis_index("c")
    s = lax.axis_index("s")

    def slab_start(p, j, b):
        rbase = ((((p * NC) + c) * NS + s) * SBT + j) * SB
        cbase = (s * SBT + j) * SB
        pltpu.async_copy(rows_hbm.at[pl.ds(rbase, SB)],
                         rows_v.at[pl.ds(b * SB, SB)], lsem)
        pltpu.async_copy(cols_hbm.at[pl.ds(cbase, SB)],
                         cols_v.at[pl.ds(b * SB, SB)], lsem)
        pltpu.async_copy(ew_hbm.at[pl.ds(cbase * G, SB * G)],
                         ew_v.at[pl.ds(b * SB * G, SB * G)], lsem)

    def slab_wait(b):
        pltpu.make_async_copy(rows_hbm.at[pl.ds(0, SB)],
                              rows_v.at[pl.ds(b * SB, SB)], lsem).wait()
        pltpu.make_async_copy(cols_hbm.at[pl.ds(0, SB)],
                              cols_v.at[pl.ds(b * SB, SB)], lsem).wait()
        pltpu.make_async_copy(ew_hbm.at[pl.ds(0, SB * G)],
                              ew_v.at[pl.ds(b * SB * G, SB * G)], lsem).wait()

    def scale(stg, scl, ewbase):
        @pl.loop(0, G // 8)
        def _sc(i8):
            for u in range(8):
                ii = i8 * 8 + u
                ewv = plsc.load_gather(
                    ew_v, [jnp.full((L,), ewbase + ii, jnp.int32)])
                for f in range(HALF // L):
                    scl[ii, pl.ds(f * L, L)] = stg[ii, pl.ds(f * L, L)] * ewv

    def gather_start(ridx, stg, sem):
        pltpu.async_copy(hs_hbm.at[rows_v.at[ridx]], stg, sem)

    def gather_wait(stg, sem):
        pltpu.make_async_copy(hs_hbm.at[rows_v.at[0]], stg, sem).wait()

    def scatter_start(cidx, scl, sem):
        pltpu.async_copy(scl, acc_sh.at[cols_v.at[cidx]], sem, add=True)

    def scatter_wait(scl, sem):
        pltpu.make_async_copy(scl, acc_sh.at[cols_v.at[0]], sem).wait()

    @pl.loop(0, K)
    def _phase(p):
        pltpu.sync_copy(zeros_hbm, acc_sh.at[pl.ds(s * NPT, NPT)])
        slab_start(p, 0, 0)
        slab_wait(0)
        plsc.subcore_barrier()

        gather_start(0, stage0, gsem0)
        gather_start(1, stage1, gsem1)

        @pl.loop(0, NSB)
        def _sblock(j):
            b = j % 2
            bn = (j + 1) % 2
            for gi in range(SB):
                stg, scl, gsem, ssem = (
                    (stage0, scaled0, gsem0, ssem0) if gi % 2 == 0 else
                    (stage1, scaled1, gsem1, ssem1))
                gather_wait(stg, gsem)
                if gi >= 2:
                    scatter_wait(scl, ssem)
                else:
                    @pl.when(j > 0)
                    def _():
                        scatter_wait(scl, ssem)
                if gi == 1:
                    slab_start(p, j + 1, bn)
                scale(stg, scl, b * SB * G + gi * G)
                if gi == 5:
                    slab_wait(bn)
                if gi < SB - 2:
                    gather_start(b * SB + gi + 2, stg, gsem)
                else:
                    gather_start(bn * SB + gi - 6, stg, gsem)
                scatter_start(b * SB + gi, scl, ssem)

        scatter_wait(scaled0, ssem0)
        scatter_wait(scaled1, ssem1)
        gather_wait(stage0, gsem0)
        gather_wait(stage1, gsem1)
        plsc.subcore_barrier()

        pltpu.sync_copy(
            acc_sh.at[pl.ds(s * NPT, NPT)],
            out_hbm.at[pl.ds((p * NC + c) * NP + s * NPT, NPT)])
        plsc.subcore_barrier()


def _prep_body(x_ref, degp_ref, rw_ref, iw_ref, dinv_ref, root_ref, hs0_ref):
    deg = jnp.sum(degp_ref[0], axis=0)
    dinv = jnp.where(deg > 0, lax.rsqrt(jnp.maximum(deg, 1e-12)), 0.0)
    dinvc = dinv[:, None]
    dinv_ref[...] = dinvc
    xb = x_ref[...]
    root_ref[0] = jnp.dot(xb, rw_ref[0], preferred_element_type=jnp.float32)
    hs = jnp.dot(xb * dinvc, iw_ref[0], preferred_element_type=jnp.float32)
    for h in range(NC):
        hs0_ref[0, h] = hs[:, h * HALF:(h + 1) * HALF]


def _prep_call(x, degp_t, rw, iw):
    return pl.pallas_call(
        _prep_body,
        grid=(K, NB),
        in_specs=[
            pl.BlockSpec((BN, F), lambda k, n: (n, 0)),
            pl.BlockSpec((1, NC * NS, BN), lambda k, n: (n, 0, 0)),
            pl.BlockSpec((1, F, F), lambda k, n: (k, 0, 0)),
            pl.BlockSpec((1, F, F), lambda k, n: (k, 0, 0)),
        ],
        out_specs=[
            pl.BlockSpec((BN, 1), lambda k, n: (n, 0)),
            pl.BlockSpec((1, BN, F), lambda k, n: (k, n, 0)),
            pl.BlockSpec((1, NC, BN, HALF), lambda k, n: (k, 0, n, 0)),
        ],
        out_shape=[
            jax.ShapeDtypeStruct((N, 1), jnp.float32),
            jax.ShapeDtypeStruct((K, N, F), jnp.float32),
            jax.ShapeDtypeStruct((K, NC, N, HALF), jnp.float32),
        ],
    )(x, degp_t, rw, iw)


def _mid_body(ss_ref, dinv_ref, root_ref, bias_ref, w_ref, hs1_ref):
    ssb = jnp.concatenate([ss_ref[0, h] for h in range(NC)], axis=1)
    dinvc = dinv_ref[...]
    o = jax.nn.relu(ssb * dinvc + root_ref[0] + bias_ref[0])
    h = jnp.dot(o * dinvc, w_ref[0], preferred_element_type=jnp.float32)
    for hh in range(NC):
        hs1_ref[0, hh] = h[:, hh * HALF:(hh + 1) * HALF]


def _mid_call(ss, dinv_c, root, bias3, w):
    return pl.pallas_call(
        _mid_body,
        grid=(K, NB),
        in_specs=[
            pl.BlockSpec((1, NC, BN, HALF), lambda k, n: (k, 0, n, 0)),
            pl.BlockSpec((BN, 1), lambda k, n: (n, 0)),
            pl.BlockSpec((1, BN, F), lambda k, n: (k, n, 0)),
            pl.BlockSpec((1, 1, F), lambda k, n: (k, 0, 0)),
            pl.BlockSpec((1, F, F), lambda k, n: (k, 0, 0)),
        ],
        out_specs=pl.BlockSpec((1, NC, BN, HALF), lambda k, n: (k, 0, n, 0)),
        out_shape=jax.ShapeDtypeStruct((K, NC, N, HALF), jnp.float32),
    )(ss, dinv_c, root, bias3, w)


def _final_body(ss_ref, dinv_ref, root_ref, bias_ref, y_ref):
    dinvc = dinv_ref[...]
    acc = jnp.zeros((BN, F), jnp.float32)
    for k in range(K):
        ssb = jnp.concatenate([ss_ref[k, h] for h in range(NC)], axis=1)
        acc += jax.nn.relu(ssb * dinvc + root_ref[k] + bias_ref[k])
    y_ref[...] = jax.nn.relu(acc * (1.0 / K))


def _final_call(ss, dinv_c, root, bias3):
    return pl.pallas_call(
        _final_body,
        grid=(NB,),
        in_specs=[
            pl.BlockSpec((K, NC, BN, HALF), lambda n: (0, 0, n, 0)),
            pl.BlockSpec((BN, 1), lambda n: (n, 0)),
            pl.BlockSpec((K, BN, F), lambda n: (0, n, 0)),
            pl.BlockSpec((K, 1, F), lambda n: (0, 0, 0)),
        ],
        out_specs=pl.BlockSpec((BN, F), lambda n: (n, 0)),
        out_shape=jax.ShapeDtypeStruct((N, F), jnp.float32),
    )(ss, dinv_c, root, bias3)


def kernel(x, edge_index, edge_weight, init_weight, weight, root_weight, bias):
    row = edge_index[0]
    col = edge_index[1]
    pad = E_PAD - E
    row_p = jnp.pad(row, (0, pad))
    col_p = jnp.pad(col, (0, pad))
    ew_p = jnp.pad(edge_weight, (0, pad))

    degp_t = _deg_kernel(col_p, ew_p).reshape(
        NC * NS, NH)[:, :N].reshape(NC * NS, NB, BN).transpose(1, 0, 2)

    bias3 = bias[0, :, 0, :][:, None, :]
    dinv_c, root, hs0 = _prep_call(x, degp_t, root_weight[0], init_weight)

    rt = jnp.pad(row_p.reshape(NS, NSB * SB, G), ((0, 0), (0, SB), (0, 0)))
    rows_off = jnp.concatenate(
        [rt + (p * NC + h) * N for p in range(K) for h in range(NC)],
        axis=0).reshape(K * NC * NS * SBT * SB, G)
    ct = jnp.pad(col_p.reshape(NS, NSB * SB, G), ((0, 0), (0, SB), (0, 0)))
    cols_g = ct.reshape(NS * SBT * SB, G)
    et = jnp.pad(ew_p.reshape(NS, NSB * SB * G), ((0, 0), (0, SB * G)))
    ew_g = et.reshape(NS * SBT * SB * G)
    zeros_z = jnp.zeros((NPT, HALF), jnp.float32)

    ss0 = _prop_kernel(hs0.reshape(K * NC * N, HALF), rows_off, cols_g,
                       ew_g, zeros_z).reshape(K, NC, NP, HALF)
    hs1 = _mid_call(ss0, dinv_c, root, bias3, weight[0])
    ss1 = _prop_kernel(hs1.reshape(K * NC * N, HALF), rows_off, cols_g,
                       ew_g, zeros_z).reshape(K, NC, NP, HALF)
    return _final_call(ss1, dinv_c, root, bias3)


# speedup vs baseline: 24.1763x; 1.4024x over previous
import functools
import jax
import jax.numpy as jnp
from jax import lax
from jax.experimental import pallas as pl
from jax.experimental.pallas import tpu as pltpu
from jax.experimental.pallas import tpu_sc as plsc

N = 10000
E = 160000
F = 256
HALF = 128
K = 2
NC = 2
NS = 16
L = 16

G = 80
SB = 8
NSB = 16
SBT = NSB + 1
NG = NSB * SB
EP = NG * G
E_PAD = NS * EP
EW_DEG = E_PAD // (NC * NS)
NH = 10240
NP = 10112
NPT = NP // NS

BN = 1000
NB = N // BN

_MESH = plsc.VectorSubcoreMesh(
    core_axis_name="c", subcore_axis_name="s", num_cores=NC, num_subcores=NS)
_SC_PARAMS = pltpu.CompilerParams(needs_layout_passes=False)


@functools.partial(
    pl.kernel,
    out_type=jax.ShapeDtypeStruct((NC * NS * NH,), jnp.float32),
    mesh=_MESH,
    scratch_types=[
        pltpu.VMEM((EW_DEG,), jnp.int32),
        pltpu.VMEM((EW_DEG,), jnp.float32),
        pltpu.VMEM((NH,), jnp.float32),
    ],
    compiler_params=_SC_PARAMS,
)
def _deg_kernel(col_hbm, ew_hbm, out_hbm, col_v, ew_v, acc_v):
    c = lax.axis_index("c")
    s = lax.axis_index("s")
    wid = c * NS + s
    base = wid * EW_DEG
    pltpu.sync_copy(col_hbm.at[pl.ds(base, EW_DEG)], col_v)
    pltpu.sync_copy(ew_hbm.at[pl.ds(base, EW_DEG)], ew_v)

    zeros = jnp.zeros((L,), jnp.float32)

    @pl.loop(0, NH // L)
    def _zero(i):
        acc_v[pl.ds(i * L, L)] = zeros

    @pl.loop(0, EW_DEG // L)
    def _accum(g):
        cv = col_v[pl.ds(g * L, L)]
        wv = ew_v[pl.ds(g * L, L)]
        plsc.addupdate_scatter(acc_v, [cv], wv)

    pltpu.sync_copy(acc_v, out_hbm.at[pl.ds(wid * NH, NH)])


@functools.partial(
    pl.kernel,
    out_type=jax.ShapeDtypeStruct((K * NC * NP, HALF), jnp.float32),
    mesh=_MESH,
    scratch_types=[
        pltpu.VMEM((2 * SB, G), jnp.int32),
        pltpu.VMEM((2 * SB, G), jnp.int32),
        pltpu.VMEM((2 * SB * G,), jnp.float32),
        pltpu.VMEM((G, HALF), jnp.float32),
        pltpu.VMEM((G, HALF), jnp.float32),
        pltpu.VMEM((G, HALF), jnp.float32),
        pltpu.VMEM((G, HALF), jnp.float32),
        pltpu.VMEM_SHARED((NP, HALF), jnp.float32),
        pltpu.SemaphoreType.DMA,
        pltpu.SemaphoreType.DMA,
        pltpu.SemaphoreType.DMA,
        pltpu.SemaphoreType.DMA,
        pltpu.SemaphoreType.DMA,
    ],
    compiler_params=_SC_PARAMS,
)
def _prop_kernel(hs_hbm, rows_hbm, cols_hbm, ew_hbm, zeros_hbm, out_hbm,
                 rows_v, cols_v, ew_v, stage0, stage1, scaled0, scaled1,
                 acc_sh, gsem0, gsem1, ssem0, ssem1, lsem):
    c = lax.axis_index("c")
    s = lax.axis_index("s")

    def slab_start(p, j, b):
        rbase = ((((p * NC) + c) * NS + s) * SBT + j) * SB
        cbase = (s * SBT + j) * SB
        pltpu.async_copy(rows_hbm.at[pl.ds(rbase, SB)],
                         rows_v.at[pl.ds(b * SB, SB)], lsem)
        pltpu.async_copy(cols_hbm.at[pl.ds(cbase, SB)],
                         cols_v.at[pl.ds(b * SB, SB)], lsem)
        pltpu.async_copy(ew_hbm.at[pl.ds(cbase * G, SB * G)],
                         ew_v.at[pl.ds(b * SB * G, SB * G)], lsem)

    def slab_wait(b):
        pltpu.make_async_copy(rows_hbm.at[pl.ds(0, SB)],
                              rows_v.at[pl.ds(b * SB, SB)], lsem).wait()
        pltpu.make_async_copy(cols_hbm.at[pl.ds(0, SB)],
                              cols_v.at[pl.ds(b * SB, SB)], lsem).wait()
        pltpu.make_async_copy(ew_hbm.at[pl.ds(0, SB * G)],
                              ew_v.at[pl.ds(b * SB * G, SB * G)], lsem).wait()

    def scale(stg, scl, ewbase):
        @pl.loop(0, G // 2)
        def _pair(ip):
            i0 = ip * 2
            i1 = i0 + 1
            b0 = plsc.load_gather(
                ew_v, [jnp.full((L,), ewbase + i0, jnp.int32)])
            b1 = plsc.load_gather(
                ew_v, [jnp.full((L,), ewbase + i1, jnp.int32)])
            nf = HALF // L
            l0 = [stg[i0, pl.ds(f * L, L)] for f in range(nf)]
            l1 = [stg[i1, pl.ds(f * L, L)] for f in range(nf)]
            for f in range(nf):
                scl[i0, pl.ds(f * L, L)] = l0[f] * b0
            for f in range(nf):
                scl[i1, pl.ds(f * L, L)] = l1[f] * b1

    def gather_start(ridx, stg, sem):
        pltpu.async_copy(hs_hbm.at[rows_v.at[ridx]], stg, sem)

    def gather_wait(stg, sem):
        pltpu.make_async_copy(hs_hbm.at[rows_v.at[0]], stg, sem).wait()

    def scatter_start(cidx, scl, sem):
        pltpu.async_copy(scl, acc_sh.at[cols_v.at[cidx]], sem, add=True)

    def scatter_wait(scl, sem):
        pltpu.make_async_copy(scl, acc_sh.at[cols_v.at[0]], sem).wait()

    @pl.loop(0, K)
    def _phase(p):
        pltpu.sync_copy(zeros_hbm, acc_sh.at[pl.ds(s * NPT, NPT)])
        slab_start(p, 0, 0)
        slab_wait(0)
        plsc.subcore_barrier()

        gather_start(0, stage0, gsem0)
        gather_start(1, stage1, gsem1)

        @pl.loop(0, NSB)
        def _sblock(j):
            b = j % 2
            bn = (j + 1) % 2
            for gi in range(SB):
                stg, scl, gsem, ssem = (
                    (stage0, scaled0, gsem0, ssem0) if gi % 2 == 0 else
                    (stage1, scaled1, gsem1, ssem1))
                gather_wait(stg, gsem)
                if gi >= 2:
                    scatter_wait(scl, ssem)
                else:
                    @pl.when(j > 0)
                    def _():
                        scatter_wait(scl, ssem)
                if gi == 1:
                    slab_start(p, j + 1, bn)
                scale(stg, scl, b * SB * G + gi * G)
                if gi == 5:
                    slab_wait(bn)
                if gi < SB - 2:
                    gather_start(b * SB + gi + 2, stg, gsem)
                else:
                    gather_start(bn * SB + gi - 6, stg, gsem)
                scatter_start(b * SB + gi, scl, ssem)

        scatter_wait(scaled0, ssem0)
        scatter_wait(scaled1, ssem1)
        gather_wait(stage0, gsem0)
        gather_wait(stage1, gsem1)
        plsc.subcore_barrier()

        pltpu.sync_copy(
            acc_sh.at[pl.ds(s * NPT, NPT)],
            out_hbm.at[pl.ds((p * NC + c) * NP + s * NPT, NPT)])
        plsc.subcore_barrier()


def _prep_body(x_ref, degp_ref, rw_ref, iw_ref, dinv_ref, root_ref, hs0_ref):
    deg = jnp.sum(degp_ref[0], axis=0)
    dinv = jnp.where(deg > 0, lax.rsqrt(jnp.maximum(deg, 1e-12)), 0.0)
    dinvc = dinv[:, None]
    dinv_ref[...] = dinvc
    xb = x_ref[...]
    root_ref[0] = jnp.dot(xb, rw_ref[0], preferred_element_type=jnp.float32)
    hs = jnp.dot(xb * dinvc, iw_ref[0], preferred_element_type=jnp.float32)
    for h in range(NC):
        hs0_ref[0, h] = hs[:, h * HALF:(h + 1) * HALF]


def _prep_call(x, degp_t, rw, iw):
    return pl.pallas_call(
        _prep_body,
        grid=(K, NB),
        in_specs=[
            pl.BlockSpec((BN, F), lambda k, n: (n, 0)),
            pl.BlockSpec((1, NC * NS, BN), lambda k, n: (n, 0, 0)),
            pl.BlockSpec((1, F, F), lambda k, n: (k, 0, 0)),
            pl.BlockSpec((1, F, F), lambda k, n: (k, 0, 0)),
        ],
        out_specs=[
            pl.BlockSpec((BN, 1), lambda k, n: (n, 0)),
            pl.BlockSpec((1, BN, F), lambda k, n: (k, n, 0)),
            pl.BlockSpec((1, NC, BN, HALF), lambda k, n: (k, 0, n, 0)),
        ],
        out_shape=[
            jax.ShapeDtypeStruct((N, 1), jnp.float32),
            jax.ShapeDtypeStruct((K, N, F), jnp.float32),
            jax.ShapeDtypeStruct((K, NC, N, HALF), jnp.float32),
        ],
    )(x, degp_t, rw, iw)


def _mid_body(ss_ref, dinv_ref, root_ref, bias_ref, w_ref, hs1_ref):
    ssb = jnp.concatenate([ss_ref[0, h] for h in range(NC)], axis=1)
    dinvc = dinv_ref[...]
    o = jax.nn.relu(ssb * dinvc + root_ref[0] + bias_ref[0])
    h = jnp.dot(o * dinvc, w_ref[0], preferred_element_type=jnp.float32)
    for hh in range(NC):
        hs1_ref[0, hh] = h[:, hh * HALF:(hh + 1) * HALF]


def _mid_call(ss, dinv_c, root, bias3, w):
    return pl.pallas_call(
        _mid_body,
        grid=(K, NB),
        in_specs=[
            pl.BlockSpec((1, NC, BN, HALF), lambda k, n: (k, 0, n, 0)),
            pl.BlockSpec((BN, 1), lambda k, n: (n, 0)),
            pl.BlockSpec((1, BN, F), lambda k, n: (k, n, 0)),
            pl.BlockSpec((1, 1, F), lambda k, n: (k, 0, 0)),
            pl.BlockSpec((1, F, F), lambda k, n: (k, 0, 0)),
        ],
        out_specs=pl.BlockSpec((1, NC, BN, HALF), lambda k, n: (k, 0, n, 0)),
        out_shape=jax.ShapeDtypeStruct((K, NC, N, HALF), jnp.float32),
    )(ss, dinv_c, root, bias3, w)


def _final_body(ss_ref, dinv_ref, root_ref, bias_ref, y_ref):
    dinvc = dinv_ref[...]
    acc = jnp.zeros((BN, F), jnp.float32)
    for k in range(K):
        ssb = jnp.concatenate([ss_ref[k, h] for h in range(NC)], axis=1)
        acc += jax.nn.relu(ssb * dinvc + root_ref[k] + bias_ref[k])
    y_ref[...] = jax.nn.relu(acc * (1.0 / K))


def _final_call(ss, dinv_c, root, bias3):
    return pl.pallas_call(
        _final_body,
        grid=(NB,),
        in_specs=[
            pl.BlockSpec((K, NC, BN, HALF), lambda n: (0, 0, n, 0)),
            pl.BlockSpec((BN, 1), lambda n: (n, 0)),
            pl.BlockSpec((K, BN, F), lambda n: (0, n, 0)),
            pl.BlockSpec((K, 1, F), lambda n: (0, 0, 0)),
        ],
        out_specs=pl.BlockSpec((BN, F), lambda n: (n, 0)),
        out_shape=jax.ShapeDtypeStruct((N, F), jnp.float32),
    )(ss, dinv_c, root, bias3)


def kernel(x, edge_index, edge_weight, init_weight, weight, root_weight, bias):
    row = edge_index[0]
    col = edge_index[1]
    pad = E_PAD - E
    row_p = jnp.pad(row, (0, pad))
    col_p = jnp.pad(col, (0, pad))
    ew_p = jnp.pad(edge_weight, (0, pad))

    degp_t = _deg_kernel(col_p, ew_p).reshape(
        NC * NS, NH)[:, :N].reshape(NC * NS, NB, BN).transpose(1, 0, 2)

    bias3 = bias[0, :, 0, :][:, None, :]
    dinv_c, root, hs0 = _prep_call(x, degp_t, root_weight[0], init_weight)

    rt = jnp.pad(row_p.reshape(NS, NSB * SB, G), ((0, 0), (0, SB), (0, 0)))
    rows_off = jnp.concatenate(
        [rt + (p * NC + h) * N for p in range(K) for h in range(NC)],
        axis=0).reshape(K * NC * NS * SBT * SB, G)
    ct = jnp.pad(col_p.reshape(NS, NSB * SB, G), ((0, 0), (0, SB), (0, 0)))
    cols_g = ct.reshape(NS * SBT * SB, G)
    et = jnp.pad(ew_p.reshape(NS, NSB * SB * G), ((0, 0), (0, SB * G)))
    ew_g = et.reshape(NS * SBT * SB * G)
    zeros_z = jnp.zeros((NPT, HALF), jnp.float32)

    ss0 = _prop_kernel(hs0.reshape(K * NC * N, HALF), rows_off, cols_g,
                       ew_g, zeros_z).reshape(K, NC, NP, HALF)
    hs1 = _mid_call(ss0, dinv_c, root, bias3, weight[0])
    ss1 = _prop_kernel(hs1.reshape(K * NC * N, HALF), rows_off, cols_g,
                       ew_g, zeros_z).reshape(K, NC, NP, HALF)
    return _final_call(ss1, dinv_c, root, bias3)
